# baseline (speedup 1.0000x reference)
import functools

import jax
import jax.numpy as jnp
from jax import lax
from jax.experimental import pallas as pl
from jax.experimental.pallas import tpu as pltpu
from jax.experimental.pallas import tpu_sc as plsc

NUM_ROWS = 1000000
EMBED_DIM = 64
BATCH = 16384
PADDED = 128

L = 16
BMAIN = (NUM_ROWS // 1024) * 1024
BTAIL = NUM_ROWS - BMAIN
TCHUNK = 16384
H = TCHUNK // 2
NBLOCKS = (NUM_ROWS + TCHUNK - 1) // TCHUNK
PACKED_ROWS = NBLOCKS * H


def _tp_body(x_ref, o_ref):
    eye = jnp.eye(EMBED_DIM, dtype=jnp.float32)
    dn = (((0,), (0,)), ((), ()))
    o_ref[:, 0:EMBED_DIM] = jax.lax.dot_general(
        x_ref[:, 0:H], eye, dn, preferred_element_type=jnp.float32)
    o_ref[:, EMBED_DIM:PADDED] = jax.lax.dot_general(
        x_ref[:, H:TCHUNK], eye, dn, preferred_element_type=jnp.float32)


_tc_transpose = pl.pallas_call(
    _tp_body,
    grid=(NBLOCKS,),
    in_specs=[pl.BlockSpec((EMBED_DIM, TCHUNK), lambda i: (0, i))],
    out_specs=pl.BlockSpec((H, PADDED), lambda i: (i, 0)),
    out_shape=jax.ShapeDtypeStruct((PACKED_ROWS, PADDED), jnp.float32),
)


def _make_sc_kernel():
    info = plsc.get_sparse_core_info()
    nc, ns = info.num_cores, info.num_subcores
    nw = nc * ns
    bpw = BATCH // nw
    half = bpw // 2
    nblk = half // L

    mesh = plsc.VectorSubcoreMesh(core_axis_name="c", subcore_axis_name="s")

    @functools.partial(
        pl.kernel,
        mesh=mesh,
        out_type=jax.ShapeDtypeStruct((BATCH,), jnp.float32),
        scratch_types=[
            pltpu.VMEM((bpw,), jnp.int32),
            pltpu.VMEM((bpw,), jnp.int32),
            pltpu.VMEM((bpw,), jnp.int32),
            pltpu.VMEM((bpw,), jnp.int32),
            pltpu.VMEM((bpw,), jnp.int32),
            pltpu.VMEM((bpw,), jnp.int32),
            pltpu.VMEM((bpw,), jnp.int32),
            pltpu.VMEM((bpw,), jnp.int32),
            pltpu.VMEM((bpw,), jnp.int32),
            pltpu.VMEM((bpw,), jnp.int32),
            pltpu.VMEM((bpw,), jnp.float32),
            pltpu.VMEM((bpw,), jnp.float32),
            pltpu.VMEM((2, half // 2, PADDED), jnp.float32),
            pltpu.VMEM((2, half // 2, PADDED), jnp.float32),
            pltpu.VMEM((bpw,), jnp.float32),
            pltpu.VMEM((bpw,), jnp.float32),
            pltpu.VMEM((L,), jnp.float32),
            pltpu.VMEM((bpw,), jnp.float32),
            pltpu.SemaphoreType.DMA,
            pltpu.SemaphoreType.DMA,
            pltpu.SemaphoreType.DMA,
        ],
        compiler_params=pltpu.CompilerParams(
            needs_layout_passes=False, use_tc_tiling_on_sc=False),
    )
    def sc_kernel(uid_hbm, eid_hbm, ut_hbm, et_hbm, ub_hbm, ubt_hbm, eb_hbm,
                  ebt_hbm, gb_hbm, out_hbm, uid_v, eid_v, idxu_v, idxe_v,
                  cbu_v, cbe_v, bmu_v, bme_v, btu_v, bte_v, ubt_v, ebt_v,
                  u_rows, e_rows, ub_v, eb_v, gb_v, scores_v,
                  sem0, sem1, semb):
        wid = lax.axis_index("s") * nc + lax.axis_index("c")
        base = wid * bpw

        pltpu.sync_copy(uid_hbm.at[pl.ds(base, bpw)], uid_v)
        pltpu.sync_copy(eid_hbm.at[pl.ds(base, bpw)], eid_v)
        pltpu.sync_copy(gb_hbm.at[pl.ds(0, 1)], gb_v.at[pl.ds(0, 1)])

        def remap(j, _):
            sl = pl.ds(j * L, L)
            u = uid_v[sl]
            idxu_v[sl] = ((u >> 14) << 13) + (u & (H - 1))
            cbu_v[sl] = ((u >> 13) & 1) << 6
            bmu_v[sl] = jnp.minimum(u, BMAIN - 1)
            btu_v[sl] = jnp.where(u >= BMAIN, u - BMAIN, u & 511)
            e = eid_v[sl]
            idxe_v[sl] = ((e >> 14) << 13) + (e & (H - 1))
            cbe_v[sl] = ((e >> 13) & 1) << 6
            bme_v[sl] = jnp.minimum(e, BMAIN - 1)
            bte_v[sl] = jnp.where(e >= BMAIN, e - BMAIN, e & 511)
            return _

        lax.fori_loop(0, bpw // L, remap, None)

        cub = pltpu.async_copy(ub_hbm.at[0].at[bmu_v], ub_v, semb)
        ceb = pltpu.async_copy(eb_hbm.at[0].at[bme_v], eb_v, semb)
        cut = pltpu.async_copy(ubt_hbm.at[0].at[btu_v], ubt_v, semb)
        cet = pltpu.async_copy(ebt_hbm.at[0].at[bte_v], ebt_v, semb)

        gb = gb_v[pl.ds(0, L)][0]
        lane = lax.iota(jnp.int32, L)

        Q = half // 2
        NQ = bpw // Q

        def fire(c):
            sl = pl.ds(c * Q, Q)
            return (pltpu.async_copy(ut_hbm.at[idxu_v.at[sl]],
                                     u_rows.at[c % 2], sem0),
                    pltpu.async_copy(et_hbm.at[idxe_v.at[sl]],
                                     e_rows.at[c % 2], sem1))

        pend = fire(0)
        for c in range(NQ):
            nxt = fire(c + 1) if c + 1 < NQ else None
            pend[0].wait()
            pend[1].wait()
            pend = nxt
            ub_ref = u_rows.at[c % 2]
            eb_ref = e_rows.at[c % 2]

            def block(j, _):
                row = jnp.full((L,), j * L, jnp.int32) + lane
                cbu = cbu_v[pl.ds(c * Q + j * L, L)]
                cbe = cbe_v[pl.ds(c * Q + j * L, L)]
                acc0 = jnp.full((L,), gb, jnp.float32)
                acc1 = jnp.zeros((L,), jnp.float32)
                for d in range(0, EMBED_DIM, 2):
                    acc0 = acc0 + (plsc.load_gather(ub_ref, [row, cbu + d])
                                   * plsc.load_gather(eb_ref, [row, cbe + d]))
                    acc1 = acc1 + (
                        plsc.load_gather(ub_ref, [row, cbu + (d + 1)])
                        * plsc.load_gather(eb_ref, [row, cbe + (d + 1)]))
                scores_v[pl.ds(c * Q + j * L, L)] = acc0 + acc1
                return _

            lax.fori_loop(0, Q // L, block, None)

        def bias_block(j, _):
            sl = pl.ds(j * L, L)
            ub = jnp.where(uid_v[sl] >= BMAIN, ubt_v[sl], ub_v[sl])
            eb = jnp.where(eid_v[sl] >= BMAIN, ebt_v[sl], eb_v[sl])
            scores_v[sl] = scores_v[sl] + ub + eb
            return _

        cub.wait()
        ceb.wait()
        cut.wait()
        cet.wait()
        lax.fori_loop(0, bpw // L, bias_block, None)
        pltpu.sync_copy(scores_v, out_hbm.at[pl.ds(base, bpw)])

    return sc_kernel


_sc_kernel = _make_sc_kernel()


def kernel(user_ids, event_ids, user_table, event_table, user_bias,
           event_bias, global_bias):
    uid = user_ids.astype(jnp.int32)
    eid = event_ids.astype(jnp.int32)
    ut2 = _tc_transpose(user_table.T)
    et2 = _tc_transpose(event_table.T)
    ub = user_bias.reshape(1, -1)
    eb = event_bias.reshape(1, -1)
    return _sc_kernel(uid, eid, ut2, et2,
                      ub[:, :BMAIN], ub[:, BMAIN:],
                      eb[:, :BMAIN], eb[:, BMAIN:],
                      global_bias)


# speedup vs baseline: 1.0707x; 1.0707x over previous
import functools

import jax
import jax.numpy as jnp
from jax import lax
from jax.experimental import pallas as pl
from jax.experimental.pallas import tpu as pltpu
from jax.experimental.pallas import tpu_sc as plsc

NUM_ROWS = 1000000
EMBED_DIM = 64
BATCH = 16384
PADDED = 128

L = 16
BMAIN = (NUM_ROWS // 1024) * 1024
BTAIL = NUM_ROWS - BMAIN
TCHUNK = 16384
H = TCHUNK // 2
NBLOCKS = (NUM_ROWS + TCHUNK - 1) // TCHUNK
PACKED_ROWS = NBLOCKS * H


def _tp_body(u_ref, e_ref, ou_ref, oe_ref):
    eye = jnp.eye(EMBED_DIM, dtype=jnp.float32)
    dn = (((0,), (0,)), ((), ()))
    for x_ref, o_ref in ((u_ref, ou_ref), (e_ref, oe_ref)):
        o_ref[:, 0:EMBED_DIM] = jax.lax.dot_general(
            x_ref[:, 0:H], eye, dn, preferred_element_type=jnp.float32)
        o_ref[:, EMBED_DIM:PADDED] = jax.lax.dot_general(
            x_ref[:, H:TCHUNK], eye, dn, preferred_element_type=jnp.float32)


_tc_transpose = pl.pallas_call(
    _tp_body,
    grid=(NBLOCKS,),
    in_specs=[pl.BlockSpec((EMBED_DIM, TCHUNK), lambda i: (0, i))] * 2,
    out_specs=[pl.BlockSpec((H, PADDED), lambda i: (i, 0))] * 2,
    out_shape=[jax.ShapeDtypeStruct((PACKED_ROWS, PADDED), jnp.float32)] * 2,
)


def _make_sc_kernel():
    info = plsc.get_sparse_core_info()
    nc, ns = info.num_cores, info.num_subcores
    nw = nc * ns
    bpw = BATCH // nw
    half = bpw // 2
    nblk = half // L

    mesh = plsc.VectorSubcoreMesh(core_axis_name="c", subcore_axis_name="s")

    @functools.partial(
        pl.kernel,
        mesh=mesh,
        out_type=jax.ShapeDtypeStruct((BATCH,), jnp.float32),
        scratch_types=[
            pltpu.VMEM((bpw,), jnp.int32),
            pltpu.VMEM((bpw,), jnp.int32),
            pltpu.VMEM((bpw,), jnp.int32),
            pltpu.VMEM((bpw,), jnp.int32),
            pltpu.VMEM((bpw,), jnp.int32),
            pltpu.VMEM((bpw,), jnp.int32),
            pltpu.VMEM((bpw,), jnp.int32),
            pltpu.VMEM((bpw,), jnp.int32),
            pltpu.VMEM((bpw,), jnp.int32),
            pltpu.VMEM((bpw,), jnp.int32),
            pltpu.VMEM((bpw,), jnp.float32),
            pltpu.VMEM((bpw,), jnp.float32),
            pltpu.VMEM((2, half // 2, PADDED), jnp.float32),
            pltpu.VMEM((2, half // 2, PADDED), jnp.float32),
            pltpu.VMEM((bpw,), jnp.float32),
            pltpu.VMEM((bpw,), jnp.float32),
            pltpu.VMEM((L,), jnp.float32),
            pltpu.VMEM((bpw,), jnp.float32),
            pltpu.SemaphoreType.DMA,
            pltpu.SemaphoreType.DMA,
            pltpu.SemaphoreType.DMA,
        ],
        compiler_params=pltpu.CompilerParams(
            needs_layout_passes=False, use_tc_tiling_on_sc=False),
    )
    def sc_kernel(uid_hbm, eid_hbm, ut_hbm, et_hbm, ub_hbm, ubt_hbm, eb_hbm,
                  ebt_hbm, gb_hbm, out_hbm, uid_v, eid_v, idxu_v, idxe_v,
                  cbu_v, cbe_v, bmu_v, bme_v, btu_v, bte_v, ubt_v, ebt_v,
                  u_rows, e_rows, ub_v, eb_v, gb_v, scores_v,
                  sem0, sem1, semb):
        wid = lax.axis_index("s") * nc + lax.axis_index("c")
        base = wid * bpw

        pltpu.sync_copy(uid_hbm.at[pl.ds(base, bpw)], uid_v)
        pltpu.sync_copy(eid_hbm.at[pl.ds(base, bpw)], eid_v)
        pltpu.sync_copy(gb_hbm.at[pl.ds(0, 1)], gb_v.at[pl.ds(0, 1)])

        def remap(j, _):
            sl = pl.ds(j * L, L)
            u = uid_v[sl]
            idxu_v[sl] = ((u >> 14) << 13) + (u & (H - 1))
            cbu_v[sl] = ((u >> 13) & 1) << 6
            bmu_v[sl] = jnp.minimum(u, BMAIN - 1)
            btu_v[sl] = jnp.where(u >= BMAIN, u - BMAIN, u & 511)
            e = eid_v[sl]
            idxe_v[sl] = ((e >> 14) << 13) + (e & (H - 1))
            cbe_v[sl] = ((e >> 13) & 1) << 6
            bme_v[sl] = jnp.minimum(e, BMAIN - 1)
            bte_v[sl] = jnp.where(e >= BMAIN, e - BMAIN, e & 511)
            return _

        lax.fori_loop(0, bpw // L, remap, None)

        cub = pltpu.async_copy(ub_hbm.at[0].at[bmu_v], ub_v, semb)
        ceb = pltpu.async_copy(eb_hbm.at[0].at[bme_v], eb_v, semb)
        cut = pltpu.async_copy(ubt_hbm.at[0].at[btu_v], ubt_v, semb)
        cet = pltpu.async_copy(ebt_hbm.at[0].at[bte_v], ebt_v, semb)

        gb = gb_v[pl.ds(0, L)][0]
        lane = lax.iota(jnp.int32, L)

        Q = half // 2
        NQ = bpw // Q

        def fire(c):
            sl = pl.ds(c * Q, Q)
            return (pltpu.async_copy(ut_hbm.at[idxu_v.at[sl]],
                                     u_rows.at[c % 2], sem0),
                    pltpu.async_copy(et_hbm.at[idxe_v.at[sl]],
                                     e_rows.at[c % 2], sem1))

        pend = fire(0)
        for c in range(NQ):
            nxt = fire(c + 1) if c + 1 < NQ else None
            pend[0].wait()
            pend[1].wait()
            pend = nxt
            ub_ref = u_rows.at[c % 2]
            eb_ref = e_rows.at[c % 2]

            def block(j, _):
                row = jnp.full((L,), j * L, jnp.int32) + lane
                cbu = cbu_v[pl.ds(c * Q + j * L, L)]
                cbe = cbe_v[pl.ds(c * Q + j * L, L)]
                acc0 = jnp.full((L,), gb, jnp.float32)
                acc1 = jnp.zeros((L,), jnp.float32)
                for d in range(0, EMBED_DIM, 2):
                    acc0 = acc0 + (plsc.load_gather(ub_ref, [row, cbu + d])
                                   * plsc.load_gather(eb_ref, [row, cbe + d]))
                    acc1 = acc1 + (
                        plsc.load_gather(ub_ref, [row, cbu + (d + 1)])
                        * plsc.load_gather(eb_ref, [row, cbe + (d + 1)]))
                scores_v[pl.ds(c * Q + j * L, L)] = acc0 + acc1
                return _

            lax.fori_loop(0, Q // L, block, None)

        def bias_block(j, _):
            sl = pl.ds(j * L, L)
            ub = jnp.where(uid_v[sl] >= BMAIN, ubt_v[sl], ub_v[sl])
            eb = jnp.where(eid_v[sl] >= BMAIN, ebt_v[sl], eb_v[sl])
            scores_v[sl] = scores_v[sl] + ub + eb
            return _

        cub.wait()
        ceb.wait()
        cut.wait()
        cet.wait()
        lax.fori_loop(0, bpw // L, bias_block, None)
        pltpu.sync_copy(scores_v, out_hbm.at[pl.ds(base, bpw)])

    return sc_kernel


_sc_kernel = _make_sc_kernel()


def kernel(user_ids, event_ids, user_table, event_table, user_bias,
           event_bias, global_bias):
    uid = user_ids.astype(jnp.int32)
    eid = event_ids.astype(jnp.int32)
    ut2, et2 = _tc_transpose(user_table.T, event_table.T)
    ub = user_bias.reshape(1, -1)
    eb = event_bias.reshape(1, -1)
    return _sc_kernel(uid, eid, ut2, et2,
                      ub[:, :BMAIN], ub[:, BMAIN:],
                      eb[:, :BMAIN], eb[:, BMAIN:],
                      global_bias)
